# baseline (speedup 1.0000x reference)
import functools

import jax
import jax.numpy as jnp
from jax import lax
from jax.experimental import pallas as pl
from jax.experimental.pallas import tpu as pltpu
from jax.experimental.pallas import tpu_sc as plsc

B = 16384
DIM = 64
N = 1_000_000
NC = 2
NS = 16
NW = NC * NS
CHUNK = 512
NCHK = 62
NCHKTOT = NW * NCHK
CBL = 2048
SB = 512
TAIL = N - N % CHUNK


def _sc_stream_gather(rs_u, bs_u, cb_u, tTu, rs_i, bs_i, cb_i, tTi):
    mesh = plsc.VectorSubcoreMesh(core_axis_name="c", subcore_axis_name="s")

    @functools.partial(
        pl.kernel,
        out_type=[
            jax.ShapeDtypeStruct((B, 128), jnp.float32),
            jax.ShapeDtypeStruct((B, 128), jnp.float32),
        ],
        mesh=mesh,
        scratch_types=[
            pltpu.VMEM((64, CHUNK), jnp.float32),
            pltpu.VMEM((SB,), jnp.int32),
            pltpu.VMEM((SB,), jnp.int32),
            pltpu.VMEM((CBL,), jnp.int32),
            pltpu.VMEM((16, 128), jnp.float32),
            pltpu.SemaphoreType.DMA,
        ],
        compiler_params=pltpu.CompilerParams(
            use_tc_tiling_on_sc=True, needs_layout_passes=False),
    )
    def k(rsu_h, bsu_h, cbu_h, tTu_h, tlu_h, rsi_h, bsi_h, cbi_h, tTi_h,
          tli_h, uout_h, iout_h, chunk_v, rs_v, bs_v, cb_v, rows_v, sem):
        wid = lax.axis_index("s") * NC + lax.axis_index("c")
        iota16 = lax.broadcasted_iota(jnp.int32, (16,), 0)

        def scalar_at(pos):
            vals = plsc.load_gather(cb_v, [jnp.full((16,), pos, jnp.int32)])
            return jnp.max(vals)

        def one_table(rs_h, bs_h, cb_h, tT_h, tail_h, out_h):
            pltpu.sync_copy(cb_h, cb_v)

            def chunk_body(kk, _):
                cid = wid * NCHK + kk
                lo = scalar_at(cid)
                hi = scalar_at(cid + 1)

                @pl.when(hi > lo)
                def _():
                    clo = cid * CHUNK
                    is_tail = clo >= TAIL

                    @pl.when(jnp.logical_not(is_tail))
                    def _():
                        coff = pl.multiple_of(
                            jnp.minimum(clo, N - CHUNK), 128)
                        pltpu.sync_copy(
                            tT_h.at[:, pl.ds(coff, CHUNK)], chunk_v)

                    @pl.when(is_tail)
                    def _():
                        pltpu.sync_copy(tail_h, chunk_v)
                    lo_al = jnp.bitwise_and(lo, -8)
                    nb = (hi - lo_al + SB - 1) // SB

                    def sb_body(j, _):
                        start = pl.multiple_of(
                            jnp.minimum(lo_al + j * SB, B - SB), 8)
                        pltpu.sync_copy(rs_h.at[pl.ds(start, SB)], rs_v)
                        pltpu.sync_copy(bs_h.at[pl.ds(start, SB)], bs_v)
                        ghi = (jnp.minimum(hi, start + SB) - start + 15) // 16

                        def g_body(g, _):
                            rvec = rs_v[pl.ds(g * 16, 16)]
                            bvec = bs_v[pl.ds(g * 16, 16)]
                            inb = (rvec >= clo) & (rvec < clo + CHUNK)
                            sbase = jnp.where(is_tail, N - CHUNK, clo)
                            rrel = jnp.clip(rvec - sbase, 0, CHUNK - 1)

                            def d_body(d, _):
                                dvec = jnp.full((16,), d, jnp.int32)
                                vals = plsc.load_gather(chunk_v, [dvec, rrel])
                                plsc.store_scatter(rows_v, [iota16, dvec], vals)
                                return 0

                            lax.fori_loop(0, DIM, d_body, 0)
                            bmask = jnp.where(inb, bvec, -1)
                            pltpu.async_copy(
                                rows_v,
                                out_h.at[plsc.Indices(bmask, ignored_value=-1)],
                                sem).wait()
                            return 0

                        lax.fori_loop(0, ghi, g_body, 0)
                        return 0

                    lax.fori_loop(0, nb, sb_body, 0)
                return 0

            lax.fori_loop(0, NCHK, chunk_body, 0)

        one_table(rsu_h, bsu_h, cbu_h, tTu_h, tlu_h, uout_h)
        one_table(rsi_h, bsi_h, cbi_h, tTi_h, tli_h, iout_h)

    return k(rs_u, bs_u, cb_u, tTu, tTu[:, N - CHUNK:], rs_i, bs_i, cb_i,
             tTi, tTi[:, N - CHUNK:])


BLK = 2048


def _tc_mlp(u_rows, i_rows, w1t, b1_row, w2_row):
    def body(u_ref, i_ref, w1t_ref, b1_ref, w2_ref, out_ref):
        z = jnp.concatenate(
            [u_ref[...][:, :DIM], i_ref[...][:, :DIM]], axis=1)
        h = lax.dot_general(z, w1t_ref[...], (((1,), (0,)), ((), ())),
                            preferred_element_type=jnp.float32)
        h = jnp.maximum(h + b1_ref[...], 0.0)
        out_ref[...] = jnp.sum(h * w2_ref[...], axis=1, keepdims=True)

    out = pl.pallas_call(
        body,
        grid=(B // BLK,),
        in_specs=[
            pl.BlockSpec((BLK, 128), lambda i: (i, 0)),
            pl.BlockSpec((BLK, 128), lambda i: (i, 0)),
            pl.BlockSpec((2 * DIM, DIM), lambda i: (0, 0)),
            pl.BlockSpec((1, DIM), lambda i: (0, 0)),
            pl.BlockSpec((1, DIM), lambda i: (0, 0)),
        ],
        out_specs=pl.BlockSpec((BLK, 1), lambda i: (i, 0)),
        out_shape=jax.ShapeDtypeStruct((B, 1), jnp.float32),
    )(u_rows, i_rows, w1t, b1_row, w2_row)
    return out


def kernel(users, items, user_latent, item_latent, W1, b1, W2):
    iota = jnp.arange(B, dtype=jnp.int32)
    rs_u, bs_u = lax.sort_key_val(users, iota)
    rs_i, bs_i = lax.sort_key_val(items, iota)
    starts = jnp.arange(NCHKTOT + 1, dtype=jnp.int32) * CHUNK
    pad = jnp.full((CBL - NCHKTOT - 1,), B, jnp.int32)
    cb_u = jnp.concatenate(
        [jnp.searchsorted(rs_u, starts).astype(jnp.int32), pad])
    cb_i = jnp.concatenate(
        [jnp.searchsorted(rs_i, starts).astype(jnp.int32), pad])
    u_out, i_out = _sc_stream_gather(
        rs_u, bs_u, cb_u, user_latent.T, rs_i, bs_i, cb_i, item_latent.T)
    out = _tc_mlp(u_out, i_out, W1.T, b1.reshape(1, DIM), W2.reshape(1, DIM))
    return out.reshape(B)


# speedup vs baseline: 2.2953x; 2.2953x over previous
import functools

import jax
import jax.numpy as jnp
from jax import lax
from jax.experimental import pallas as pl
from jax.experimental.pallas import tpu as pltpu
from jax.experimental.pallas import tpu_sc as plsc

B = 16384
DIM = 64
N = 1_000_000
NC = 2
NS = 16
NW = NC * NS
CHUNK = 512
NCHK = 62
NCHKTOT = NW * NCHK
CBL = 2048
TAIL = N - N % CHUNK


def _sc_stream_gather(rs_u, bs_u, cb_u, tTu, rs_i, bs_i, cb_i, tTi):
    mesh = plsc.VectorSubcoreMesh(core_axis_name="c", subcore_axis_name="s")

    @functools.partial(
        pl.kernel,
        out_type=[
            jax.ShapeDtypeStruct((B, 128), jnp.float32),
            jax.ShapeDtypeStruct((B, 128), jnp.float32),
        ],
        mesh=mesh,
        scratch_types=[
            pltpu.VMEM((2, 64, CHUNK), jnp.float32),
            pltpu.VMEM((B,), jnp.int32),
            pltpu.VMEM((B,), jnp.int32),
            pltpu.VMEM((CBL,), jnp.int32),
            pltpu.VMEM((16, 128), jnp.float32),
            pltpu.SemaphoreType.DMA,
            pltpu.SemaphoreType.DMA,
            pltpu.SemaphoreType.DMA,
        ],
        compiler_params=pltpu.CompilerParams(
            use_tc_tiling_on_sc=True, needs_layout_passes=False),
    )
    def k(rsu_h, bsu_h, cbu_h, tTu_h, tlu_h, rsi_h, bsi_h, cbi_h, tTi_h,
          tli_h, uout_h, iout_h, chunk_v, rs_v, bs_v, cb_v, rows_v,
          sem0, sem1, osem):
        wid = lax.axis_index("s") * NC + lax.axis_index("c")
        iota16 = lax.broadcasted_iota(jnp.int32, (16,), 0)

        def scalar_at(pos):
            vals = plsc.load_gather(cb_v, [jnp.full((16,), pos, jnp.int32)])
            return jnp.max(vals)

        def one_table(rs_h, bs_h, cb_h, tT_h, tail_h, out_h):
            pltpu.sync_copy(cb_h, cb_v)
            pltpu.sync_copy(rs_h, rs_v)
            pltpu.sync_copy(bs_h, bs_v)

            def chunk_dma(kk, slot, sem):
                cid = wid * NCHK + kk
                clo = cid * CHUNK
                is_tail = clo >= TAIL

                @pl.when(jnp.logical_not(is_tail))
                def _():
                    coff = pl.multiple_of(jnp.minimum(clo, N - CHUNK), 128)
                    pltpu.async_copy(
                        tT_h.at[:, pl.ds(coff, CHUNK)], chunk_v.at[slot],
                        sem)

                @pl.when(is_tail)
                def _():
                    pltpu.async_copy(tail_h, chunk_v.at[slot], sem)

            def chunk_wait(slot, sem):
                pltpu.make_async_copy(
                    tT_h.at[:, pl.ds(0, CHUNK)], chunk_v.at[slot], sem
                ).wait()

            chunk_dma(0, 0, sem0)

            def chunk_body(kk, _):
                slot = lax.rem(kk, 2)

                @pl.when(kk + 1 < NCHK)
                def _():
                    @pl.when(slot == 0)
                    def _():
                        chunk_dma(kk + 1, 1, sem1)

                    @pl.when(slot == 1)
                    def _():
                        chunk_dma(kk + 1, 0, sem0)

                @pl.when(slot == 0)
                def _():
                    chunk_wait(0, sem0)

                @pl.when(slot == 1)
                def _():
                    chunk_wait(1, sem1)

                cid = wid * NCHK + kk
                lo = scalar_at(cid)
                hi = scalar_at(cid + 1)

                @pl.when(hi > lo)
                def _():
                    clo = cid * CHUNK
                    is_tail = clo >= TAIL
                    sbase = jnp.where(is_tail, N - CHUNK, clo)
                    glo = lax.div(lo, 16)
                    ghi = lax.div(hi + 15, 16)

                    def g_body(g, _):
                        goff = pl.multiple_of(g * 16, 8)
                        rvec = rs_v[pl.ds(goff, 16)]
                        bvec = bs_v[pl.ds(goff, 16)]
                        inb = (rvec >= clo) & (rvec < clo + CHUNK)
                        rrel = jnp.clip(rvec - sbase, 0, CHUNK - 1)

                        def d_body(d, _):
                            dvec = jnp.full((16,), d, jnp.int32)
                            vals = plsc.load_gather(
                                chunk_v.at[slot], [dvec, rrel])
                            plsc.store_scatter(rows_v, [iota16, dvec], vals)
                            return 0

                        lax.fori_loop(0, DIM, d_body, 0)
                        bmask = jnp.where(inb, bvec, -1)
                        pltpu.async_copy(
                            rows_v,
                            out_h.at[plsc.Indices(bmask, ignored_value=-1)],
                            osem).wait()
                        return 0

                    lax.fori_loop(glo, ghi, g_body, 0)
                return 0

            lax.fori_loop(0, NCHK, chunk_body, 0)

        one_table(rsu_h, bsu_h, cbu_h, tTu_h, tlu_h, uout_h)
        one_table(rsi_h, bsi_h, cbi_h, tTi_h, tli_h, iout_h)

    return k(rs_u, bs_u, cb_u, tTu, tTu[:, N - CHUNK:], rs_i, bs_i, cb_i,
             tTi, tTi[:, N - CHUNK:])


BLK = 2048


def _tc_mlp(u_rows, i_rows, w1t, b1_row, w2_row):
    def body(u_ref, i_ref, w1t_ref, b1_ref, w2_ref, out_ref):
        z = jnp.concatenate(
            [u_ref[...][:, :DIM], i_ref[...][:, :DIM]], axis=1)
        h = lax.dot_general(z, w1t_ref[...], (((1,), (0,)), ((), ())),
                            preferred_element_type=jnp.float32)
        h = jnp.maximum(h + b1_ref[...], 0.0)
        out_ref[...] = jnp.sum(h * w2_ref[...], axis=1, keepdims=True)

    out = pl.pallas_call(
        body,
        grid=(B // BLK,),
        in_specs=[
            pl.BlockSpec((BLK, 128), lambda i: (i, 0)),
            pl.BlockSpec((BLK, 128), lambda i: (i, 0)),
            pl.BlockSpec((2 * DIM, DIM), lambda i: (0, 0)),
            pl.BlockSpec((1, DIM), lambda i: (0, 0)),
            pl.BlockSpec((1, DIM), lambda i: (0, 0)),
        ],
        out_specs=pl.BlockSpec((BLK, 1), lambda i: (i, 0)),
        out_shape=jax.ShapeDtypeStruct((B, 1), jnp.float32),
    )(u_rows, i_rows, w1t, b1_row, w2_row)
    return out


def kernel(users, items, user_latent, item_latent, W1, b1, W2):
    iota = jnp.arange(B, dtype=jnp.int32)
    rs_u, bs_u = lax.sort_key_val(users, iota)
    rs_i, bs_i = lax.sort_key_val(items, iota)
    starts = jnp.arange(NCHKTOT + 1, dtype=jnp.int32) * CHUNK
    pad = jnp.full((CBL - NCHKTOT - 1,), B, jnp.int32)
    cb_u = jnp.concatenate(
        [jnp.searchsorted(rs_u, starts).astype(jnp.int32), pad])
    cb_i = jnp.concatenate(
        [jnp.searchsorted(rs_i, starts).astype(jnp.int32), pad])
    u_out, i_out = _sc_stream_gather(
        rs_u, bs_u, cb_u, user_latent.T, rs_i, bs_i, cb_i, item_latent.T)
    out = _tc_mlp(u_out, i_out, W1.T, b1.reshape(1, DIM), W2.reshape(1, DIM))
    return out.reshape(B)
